# Initial kernel scaffold; baseline (speedup 1.0000x reference)
import jax
import jax.numpy as jnp
from jax.experimental import pallas as pl


def kernel(block_features, block_onehot, output_shape):
    raise NotImplementedError("write your pallas kernel here")


# speedup vs baseline: 4.8805x; 4.8805x over previous
import functools

import jax
import jax.numpy as jnp
from jax import lax
from jax.experimental import pallas as pl
from jax.experimental.pallas import tpu as pltpu
from jax.experimental.pallas import tpu_sc as plsc

_NC = 2
_NS = 16
_NW = _NC * _NS
_LANES = 16
_SUBL = 8
_TLANE = 128


def _index_kernel(ng_max, oh_ref, widx_ref, valid_ref):
    oh = oh_ref[0]
    g_dim, n_dim = oh.shape
    m = oh > 0.0
    x = m.astype(jnp.int32)
    lanes = lax.broadcasted_iota(jnp.int32, (g_dim, n_dim), 1)
    k = 1
    while k < n_dim:
        shifted = pltpu.roll(x, k, axis=1)
        x = x + jnp.where(lanes >= k, shifted, 0)
        k *= 2
    rank = jnp.clip(x - 1, 0, ng_max - 1)
    wg = jnp.full((1, n_dim), -1, jnp.int32)
    wr = jnp.zeros((1, n_dim), jnp.int32)
    for g in range(g_dim):
        mg = m[g:g + 1]
        wg = jnp.where(mg, g, wg)
        wr = jnp.where(mg, rank[g:g + 1], wr)
    valid = wg >= 0
    g_eff = jnp.where(valid, wg, 0)
    widx_ref[0] = g_eff * ng_max + wr
    valid_ref[0] = valid.astype(jnp.float32)


def _sc_gather(x6, widx_hbm, valid_hbm, out5, tab_v, outds_v, widx_v,
               valid_v, sem, osem):
    B, G, DT, NT, DS, L = x6.shape
    n_dim = NT * L
    wid = lax.axis_index("s") * _NC + lax.axis_index("c")
    b = wid // DT
    dt = wid % DT
    pltpu.sync_copy(widx_hbm.at[b, 0], widx_v)
    pltpu.sync_copy(valid_hbm.at[b, 0], valid_v)

    def pop_body(t, acc):
        for u in range(8):
            acc = acc + valid_v[pl.ds((t * 8 + u) * _LANES, _LANES)]
        return acc
    s = lax.fori_loop(0, n_dim // (8 * _LANES), pop_body,
                      jnp.zeros((_LANES,), jnp.float32))
    total_valid = s[0]
    for e in range(1, _LANES):
        total_valid = total_valid + s[e]
    has_invalid = total_valid < float(n_dim)

    out_copies = [None, None]
    for ds in range(DS):
        stage = [pltpu.make_async_copy(x6.at[b, g, dt, :, ds, :],
                                       tab_v.at[g], sem) for g in range(G)]
        for c in stage:
            c.start()
        for c in stage:
            c.wait()

        buf = ds % 2
        if out_copies[buf] is not None:
            out_copies[buf].wait()

        def gather_body(c, carry, _buf=buf):
            for u in range(L // _LANES):
                off = c * L + u * _LANES
                wv = widx_v[pl.ds(off, _LANES)]
                gi = lax.shift_right_logical(wv, n_dim.bit_length() - 1)
                rest = lax.bitwise_and(wv, n_dim - 1)
                nti = lax.shift_right_logical(rest, L.bit_length() - 1)
                li = lax.bitwise_and(rest, L - 1)
                vals = plsc.load_gather(tab_v, [gi, nti, li])
                outds_v[_buf, c, pl.ds(u * _LANES, _LANES)] = vals
            return carry
        lax.fori_loop(0, NT, gather_body, 0)

        @pl.when(has_invalid)
        def _fix_invalid():
            def fix_body(c, carry, _buf=buf):
                for u in range(L // _LANES):
                    off = c * L + u * _LANES
                    sl = pl.ds(u * _LANES, _LANES)
                    outds_v[_buf, c, sl] = (
                        outds_v[_buf, c, sl] * valid_v[pl.ds(off, _LANES)])
                return carry
            lax.fori_loop(0, NT, fix_body, 0)

        oc = pltpu.make_async_copy(outds_v.at[buf],
                                   out5.at[b, dt, :, ds, :], osem)
        oc.start()
        out_copies[buf] = oc
    for oc in out_copies:
        if oc is not None:
            oc.wait()


def kernel(block_features, block_onehot, output_shape):
    B, G, Ng_max, D = block_features.shape
    if block_onehot.ndim == 2:
        block_onehot = block_onehot[None, :, :]
    if block_onehot.shape[0] != B:
        block_onehot = jnp.tile(block_onehot, (B, 1, 1))
    N = block_onehot.shape[1]
    DT, DS, NT, L = D // _SUBL, _SUBL, Ng_max // _TLANE, _TLANE

    oh_t = jnp.transpose(block_onehot, (0, 2, 1))
    widx, valid = pl.pallas_call(
        functools.partial(_index_kernel, Ng_max),
        grid=(B,),
        in_specs=[pl.BlockSpec((1, G, N), lambda b: (b, 0, 0))],
        out_specs=[pl.BlockSpec((1, 1, N), lambda b: (b, 0, 0)),
                   pl.BlockSpec((1, 1, N), lambda b: (b, 0, 0))],
        out_shape=[jax.ShapeDtypeStruct((B, 1, N), jnp.int32),
                   jax.ShapeDtypeStruct((B, 1, N), jnp.float32)],
    )(oh_t)

    x6 = (block_features.transpose(0, 1, 3, 2)
          .reshape(B, G, DT, DS, NT, L)
          .transpose(0, 1, 2, 4, 3, 5))

    mesh = plsc.VectorSubcoreMesh(core_axis_name="c", subcore_axis_name="s",
                                  num_cores=_NC, num_subcores=_NS)
    out5 = pl.kernel(
        _sc_gather,
        out_type=jax.ShapeDtypeStruct((B, DT, NT, DS, L), jnp.float32),
        mesh=mesh,
        compiler_params=pltpu.CompilerParams(use_tc_tiling_on_sc=False,
                                             needs_layout_passes=False),
        scratch_types=[
            pltpu.VMEM((G, NT, L), jnp.float32),
            pltpu.VMEM((2, NT, L), jnp.float32),
            pltpu.VMEM((N,), jnp.int32),
            pltpu.VMEM((N,), jnp.float32),
            pltpu.SemaphoreType.DMA,
            pltpu.SemaphoreType.DMA,
        ],
    )(x6, widx, valid)
    return out5.transpose(0, 2, 4, 1, 3).reshape(B, N, D)


# speedup vs baseline: 6.7985x; 1.3930x over previous
import functools

import jax
import jax.numpy as jnp
from jax import lax
from jax.experimental import pallas as pl
from jax.experimental.pallas import tpu as pltpu
from jax.experimental.pallas import tpu_sc as plsc

_NC = 2
_NS = 16
_NW = _NC * _NS
_LANES = 16
_SUBL = 8
_TLANE = 128


def _index_kernel(ng_max, oh_ref, widx_ref):
    oh = oh_ref[0]
    g_dim, n_dim = oh.shape
    m = oh > 0.0
    x = m.astype(jnp.int32)
    lanes = lax.broadcasted_iota(jnp.int32, (g_dim, n_dim), 1)
    k = 1
    while k < n_dim:
        shifted = pltpu.roll(x, k, axis=1)
        x = x + jnp.where(lanes >= k, shifted, 0)
        k *= 2
    rank = jnp.clip(x - 1, 0, ng_max - 1)
    wg = jnp.full((1, n_dim), -1, jnp.int32)
    wr = jnp.zeros((1, n_dim), jnp.int32)
    for g in range(g_dim):
        mg = m[g:g + 1]
        wg = jnp.where(mg, g, wg)
        wr = jnp.where(mg, rank[g:g + 1], wr)
    valid = wg >= 0
    widx_ref[0] = jnp.where(valid, wg * ng_max + wr, g_dim * ng_max)


def _sc_gather(x6, widx_hbm, out5, tab_v, outds_v, widx_v, sem, osem):
    B, G, DT, NT, DS, L = x6.shape
    n_dim = NT * L
    g_shift = n_dim.bit_length() - 1
    l_shift = L.bit_length() - 1
    wid = lax.axis_index("s") * _NC + lax.axis_index("c")
    b = wid // DT
    dt = wid % DT
    pltpu.sync_copy(widx_hbm.at[b, 0], widx_v)

    one = jnp.ones((_LANES,), jnp.int32)

    def scan_body(t, acc):
        for u in range(8):
            wv = widx_v[pl.ds((t * 8 + u) * _LANES, _LANES)]
            acc = acc | lax.shift_left(
                one, lax.shift_right_logical(wv, g_shift))
        return acc
    acc = lax.fori_loop(0, n_dim // (8 * _LANES), scan_body,
                        jnp.zeros((_LANES,), jnp.int32))
    bits = acc[0]
    for e in range(1, _LANES):
        bits = bits | acc[e]
    has_invalid = lax.bitwise_and(
        lax.shift_right_logical(bits, G), 1) != 0

    out_copies = [None, None]
    for ds in range(DS):
        stage = [pltpu.make_async_copy(x6.at[b, g, dt, :, ds, :],
                                       tab_v.at[g], sem) for g in range(G)]
        for g in range(G):
            @pl.when(lax.bitwise_and(lax.shift_right_logical(bits, g), 1)
                     != 0)
            def _start(_c=stage[g]):
                _c.start()
        for g in range(G):
            @pl.when(lax.bitwise_and(lax.shift_right_logical(bits, g), 1)
                     != 0)
            def _wait(_c=stage[g]):
                _c.wait()

        buf = ds % 2
        if out_copies[buf] is not None:
            out_copies[buf].wait()

        def gather_body(c, carry, _buf=buf):
            for u in range(L // _LANES):
                off = c * L + u * _LANES
                wv = widx_v[pl.ds(off, _LANES)]
                gi = lax.bitwise_and(
                    lax.shift_right_logical(wv, g_shift), G - 1)
                rest = lax.bitwise_and(wv, n_dim - 1)
                nti = lax.shift_right_logical(rest, l_shift)
                li = lax.bitwise_and(rest, L - 1)
                vals = plsc.load_gather(tab_v, [gi, nti, li])
                outds_v[_buf, c, pl.ds(u * _LANES, _LANES)] = vals
            return carry
        lax.fori_loop(0, NT, gather_body, 0)

        @pl.when(has_invalid)
        def _fix_invalid():
            def fix_body(c, carry, _buf=buf):
                for u in range(L // _LANES):
                    off = c * L + u * _LANES
                    wv = widx_v[pl.ds(off, _LANES)]
                    sl = pl.ds(u * _LANES, _LANES)
                    outds_v[_buf, c, sl] = jnp.where(
                        wv < G * n_dim, outds_v[_buf, c, sl], 0.0)
                return carry
            lax.fori_loop(0, NT, fix_body, 0)

        oc = pltpu.make_async_copy(outds_v.at[buf],
                                   out5.at[b, dt, :, ds, :], osem)
        oc.start()
        out_copies[buf] = oc
    for oc in out_copies:
        if oc is not None:
            oc.wait()


def kernel(block_features, block_onehot, output_shape):
    B, G, Ng_max, D = block_features.shape
    if block_onehot.ndim == 2:
        block_onehot = block_onehot[None, :, :]
    if block_onehot.shape[0] != B:
        block_onehot = jnp.tile(block_onehot, (B, 1, 1))
    N = block_onehot.shape[1]
    DT, DS, NT, L = D // _SUBL, _SUBL, Ng_max // _TLANE, _TLANE

    oh_t = jnp.transpose(block_onehot, (0, 2, 1))
    widx = pl.pallas_call(
        functools.partial(_index_kernel, Ng_max),
        grid=(B,),
        in_specs=[pl.BlockSpec((1, G, N), lambda b: (b, 0, 0))],
        out_specs=pl.BlockSpec((1, 1, N), lambda b: (b, 0, 0)),
        out_shape=jax.ShapeDtypeStruct((B, 1, N), jnp.int32),
    )(oh_t)

    x6 = (block_features.transpose(0, 1, 3, 2)
          .reshape(B, G, DT, DS, NT, L)
          .transpose(0, 1, 2, 4, 3, 5))

    mesh = plsc.VectorSubcoreMesh(core_axis_name="c", subcore_axis_name="s",
                                  num_cores=_NC, num_subcores=_NS)
    out5 = pl.kernel(
        _sc_gather,
        out_type=jax.ShapeDtypeStruct((B, DT, NT, DS, L), jnp.float32),
        mesh=mesh,
        compiler_params=pltpu.CompilerParams(use_tc_tiling_on_sc=False,
                                             needs_layout_passes=False),
        scratch_types=[
            pltpu.VMEM((G, NT, L), jnp.float32),
            pltpu.VMEM((2, NT, L), jnp.float32),
            pltpu.VMEM((N,), jnp.int32),
            pltpu.SemaphoreType.DMA,
            pltpu.SemaphoreType.DMA,
        ],
    )(x6, widx)
    return out5.transpose(0, 2, 4, 1, 3).reshape(B, N, D)


# speedup vs baseline: 7.1134x; 1.0463x over previous
import functools

import jax
import jax.numpy as jnp
from jax import lax
from jax.experimental import pallas as pl
from jax.experimental.pallas import tpu as pltpu
from jax.experimental.pallas import tpu_sc as plsc

_NC = 2
_NS = 16
_NW = _NC * _NS
_LANES = 16
_SUBL = 8
_TLANE = 128


def _index_kernel(ng_max, oh_ref, out_ref):
    oh = oh_ref[0]
    g_dim, n_dim = oh.shape
    m = oh > 0.0
    x = m.astype(jnp.int32)
    lanes = lax.broadcasted_iota(jnp.int32, (g_dim, n_dim), 1)
    k = 1
    while k < n_dim:
        shifted = pltpu.roll(x, k, axis=1)
        x = x + jnp.where(lanes >= k, shifted, 0)
        k *= 2
    rank = jnp.clip(x - 1, 0, ng_max - 1)
    wg = jnp.full((1, n_dim), -1, jnp.int32)
    wr = jnp.zeros((1, n_dim), jnp.int32)
    for g in range(g_dim):
        mg = m[g:g + 1]
        wg = jnp.where(mg, g, wg)
        wr = jnp.where(mg, rank[g:g + 1], wr)
    valid = wg >= 0
    widx = jnp.where(valid, wg * ng_max + wr, g_dim * ng_max)
    bits = jnp.max((wg == 0).astype(jnp.int32))
    for g in range(1, g_dim):
        bits = bits | (jnp.max((wg == g).astype(jnp.int32)) << g)
    bits = bits | (jnp.max((~valid).astype(jnp.int32)) << g_dim)
    out_ref[:, 0:1, :] = widx[:, None, :]
    out_ref[:, 1:2, :] = jnp.broadcast_to(bits, (1, 1, n_dim))


def _sc_gather(x6, widx_hbm, out5, tab_v, outds_v, widx_v, bits_v,
               sem, osem):
    B, G, DT, NT, DS, L = x6.shape
    n_dim = NT * L
    g_shift = n_dim.bit_length() - 1
    l_shift = L.bit_length() - 1
    wid = lax.axis_index("s") * _NC + lax.axis_index("c")
    b = wid // DT
    dt = wid % DT

    pltpu.sync_copy(widx_hbm.at[b, 1, pl.ds(0, _LANES)], bits_v)
    bits = bits_v[pl.ds(0, _LANES)][0]
    pcnt = lax.bitwise_and(bits, 1)
    g0 = pcnt * 0
    for g in range(1, G):
        bitg = lax.bitwise_and(lax.shift_right_logical(bits, g), 1)
        pcnt = pcnt + bitg
        g0 = g0 + g * bitg
    single = pcnt == 1
    has_invalid = lax.bitwise_and(lax.shift_right_logical(bits, G), 1) != 0

    fast_stage = [pltpu.make_async_copy(x6.at[b, g0, dt, :, ds, :],
                                        tab_v.at[pl.ds(ds * NT, NT)], sem)
                  for ds in range(DS)]

    @pl.when(single)
    def _start_fast():
        for c in fast_stage:
            c.start()

    pltpu.sync_copy(widx_hbm.at[b, 0], widx_v)

    @pl.when(single)
    def _fast_path():
        for c in fast_stage:
            c.wait()
        out_copies = [None, None]
        for ds in range(DS):
            buf = ds % 2
            if out_copies[buf] is not None:
                out_copies[buf].wait()

            def gather_body(c, carry, _buf=buf, _ds=ds):
                for u in range(L // _LANES):
                    off = c * L + u * _LANES
                    wv = widx_v[pl.ds(off, _LANES)]
                    rest = lax.bitwise_and(wv, n_dim - 1)
                    nti = lax.shift_right_logical(rest, l_shift)
                    li = lax.bitwise_and(rest, L - 1)
                    row = _ds * NT + nti
                    vals = plsc.load_gather(tab_v, [row, li])
                    outds_v[_buf, c, pl.ds(u * _LANES, _LANES)] = vals
                return carry
            lax.fori_loop(0, NT, gather_body, 0)

            @pl.when(has_invalid)
            def _fix_invalid():
                def fix_body(c, carry, _buf=buf):
                    for u in range(L // _LANES):
                        off = c * L + u * _LANES
                        wv = widx_v[pl.ds(off, _LANES)]
                        sl = pl.ds(u * _LANES, _LANES)
                        outds_v[_buf, c, sl] = jnp.where(
                            wv < G * n_dim, outds_v[_buf, c, sl], 0.0)
                    return carry
                lax.fori_loop(0, NT, fix_body, 0)

            oc = pltpu.make_async_copy(outds_v.at[buf],
                                       out5.at[b, dt, :, ds, :], osem)
            oc.start()
            out_copies[buf] = oc
        for oc in out_copies:
            if oc is not None:
                oc.wait()

    @pl.when(jnp.logical_not(single))
    def _slow_path():
        out_copies = [None, None]
        for ds in range(DS):
            stage = [pltpu.make_async_copy(x6.at[b, g, dt, :, ds, :],
                                           tab_v.at[pl.ds(g * NT, NT)], sem)
                     for g in range(G)]
            for g in range(G):
                @pl.when(lax.bitwise_and(
                    lax.shift_right_logical(bits, g), 1) != 0)
                def _start(_c=stage[g]):
                    _c.start()
            for g in range(G):
                @pl.when(lax.bitwise_and(
                    lax.shift_right_logical(bits, g), 1) != 0)
                def _wait(_c=stage[g]):
                    _c.wait()

            buf = ds % 2
            if out_copies[buf] is not None:
                out_copies[buf].wait()

            def gather_body(c, carry, _buf=buf):
                for u in range(L // _LANES):
                    off = c * L + u * _LANES
                    wv = widx_v[pl.ds(off, _LANES)]
                    gi = lax.bitwise_and(
                        lax.shift_right_logical(wv, g_shift), G - 1)
                    rest = lax.bitwise_and(wv, n_dim - 1)
                    nti = lax.shift_right_logical(rest, l_shift)
                    li = lax.bitwise_and(rest, L - 1)
                    vals = plsc.load_gather(tab_v, [gi * NT + nti, li])
                    outds_v[_buf, c, pl.ds(u * _LANES, _LANES)] = vals
                return carry
            lax.fori_loop(0, NT, gather_body, 0)

            @pl.when(has_invalid)
            def _fix_invalid():
                def fix_body(c, carry, _buf=buf):
                    for u in range(L // _LANES):
                        off = c * L + u * _LANES
                        wv = widx_v[pl.ds(off, _LANES)]
                        sl = pl.ds(u * _LANES, _LANES)
                        outds_v[_buf, c, sl] = jnp.where(
                            wv < G * n_dim, outds_v[_buf, c, sl], 0.0)
                    return carry
                lax.fori_loop(0, NT, fix_body, 0)

            oc = pltpu.make_async_copy(outds_v.at[buf],
                                       out5.at[b, dt, :, ds, :], osem)
            oc.start()
            out_copies[buf] = oc
        for oc in out_copies:
            if oc is not None:
                oc.wait()


def kernel(block_features, block_onehot, output_shape):
    B, G, Ng_max, D = block_features.shape
    if block_onehot.ndim == 2:
        block_onehot = block_onehot[None, :, :]
    if block_onehot.shape[0] != B:
        block_onehot = jnp.tile(block_onehot, (B, 1, 1))
    N = block_onehot.shape[1]
    DT, DS, NT, L = D // _SUBL, _SUBL, Ng_max // _TLANE, _TLANE

    oh_t = jnp.transpose(block_onehot, (0, 2, 1))
    widx = pl.pallas_call(
        functools.partial(_index_kernel, Ng_max),
        grid=(B,),
        in_specs=[pl.BlockSpec((1, G, N), lambda b: (b, 0, 0))],
        out_specs=pl.BlockSpec((1, 2, N), lambda b: (b, 0, 0)),
        out_shape=jax.ShapeDtypeStruct((B, 2, N), jnp.int32),
    )(oh_t)

    x6 = (block_features.transpose(0, 1, 3, 2)
          .reshape(B, G, DT, DS, NT, L)
          .transpose(0, 1, 2, 4, 3, 5))

    mesh = plsc.VectorSubcoreMesh(core_axis_name="c", subcore_axis_name="s",
                                  num_cores=_NC, num_subcores=_NS)
    out5 = pl.kernel(
        _sc_gather,
        out_type=jax.ShapeDtypeStruct((B, DT, NT, DS, L), jnp.float32),
        mesh=mesh,
        compiler_params=pltpu.CompilerParams(use_tc_tiling_on_sc=False,
                                             needs_layout_passes=False),
        scratch_types=[
            pltpu.VMEM((G * NT, L), jnp.float32),
            pltpu.VMEM((2, NT, L), jnp.float32),
            pltpu.VMEM((N,), jnp.int32),
            pltpu.VMEM((_LANES,), jnp.int32),
            pltpu.SemaphoreType.DMA,
            pltpu.SemaphoreType.DMA,
        ],
    )(x6, widx)
    return out5.transpose(0, 2, 4, 1, 3).reshape(B, N, D)


# speedup vs baseline: 7.7503x; 1.0895x over previous
import functools

import jax
import jax.numpy as jnp
from jax import lax
from jax.experimental import pallas as pl
from jax.experimental.pallas import tpu as pltpu
from jax.experimental.pallas import tpu_sc as plsc

_NC = 2
_NS = 16
_NW = _NC * _NS
_LANES = 16
_SUBL = 8
_TLANE = 128


def _index_kernel(b_dim, g_dim, ng_max, oh_ref, widx_ref, bits_ref):
    oh = oh_ref[...]
    n_dim = oh.shape[1]
    m = oh > 0.0
    x = m.astype(jnp.int32)
    lanes = lax.broadcasted_iota(jnp.int32, oh.shape, 1)
    k = 1
    while k < n_dim:
        shifted = pltpu.roll(x, k, axis=1)
        x = x + jnp.where(lanes >= k, shifted, 0)
        k *= 2
    rank = jnp.clip(x - 1, 0, ng_max - 1)
    g_row = lax.bitwise_and(
        lax.broadcasted_iota(jnp.int32, oh.shape, 0), g_dim - 1)
    enc = jnp.where(m, g_row * ng_max + rank, -1)
    wmax = jnp.concatenate(
        [jnp.max(enc[b * g_dim:(b + 1) * g_dim], axis=0, keepdims=True)
         for b in range(b_dim)], axis=0)
    widx = jnp.where(wmax < 0, g_dim * ng_max, wmax)
    gi = lax.shift_right_logical(widx, ng_max.bit_length() - 1)
    bits = jnp.max((gi == 0).astype(jnp.int32), axis=1, keepdims=True)
    for g in range(1, g_dim + 1):
        bits = bits | (jnp.max((gi == g).astype(jnp.int32), axis=1,
                               keepdims=True) << g)
    widx_ref[...] = widx[:, None, :]
    bits_ref[...] = jnp.broadcast_to(bits[:, :, None], (b_dim, 1, n_dim))


def _sc_gather(x6, widx_hbm, bits_hbm, out5, tab_v, outds_v, widx_v, bits_v,
               sem, osem):
    B, G, DT, NT, DS, L = x6.shape
    n_dim = NT * L
    g_shift = n_dim.bit_length() - 1
    l_shift = L.bit_length() - 1
    wid = lax.axis_index("s") * _NC + lax.axis_index("c")
    b = wid // DT
    dt = wid % DT

    pltpu.sync_copy(bits_hbm.at[b, 0, pl.ds(0, _LANES)], bits_v)
    bits = bits_v[pl.ds(0, _LANES)][0]
    pcnt = lax.bitwise_and(bits, 1)
    g0 = pcnt * 0
    for g in range(1, G):
        bitg = lax.bitwise_and(lax.shift_right_logical(bits, g), 1)
        pcnt = pcnt + bitg
        g0 = g0 + g * bitg
    single = pcnt == 1
    has_invalid = lax.bitwise_and(lax.shift_right_logical(bits, G), 1) != 0

    fast_stage = [pltpu.make_async_copy(x6.at[b, g0, dt, :, ds, :],
                                        tab_v.at[pl.ds(ds * NT, NT)], sem)
                  for ds in range(DS)]

    @pl.when(single)
    def _start_fast():
        for c in fast_stage:
            c.start()

    pltpu.sync_copy(widx_hbm.at[b, 0], widx_v)

    @pl.when(single)
    def _fast_path():
        for c in fast_stage:
            c.wait()
        out_copies = [None, None]
        for ds in range(DS):
            buf = ds % 2
            if out_copies[buf] is not None:
                out_copies[buf].wait()

            def gather_body(c, carry, _buf=buf, _ds=ds):
                for u in range(L // _LANES):
                    off = c * L + u * _LANES
                    wv = widx_v[pl.ds(off, _LANES)]
                    rest = lax.bitwise_and(wv, n_dim - 1)
                    nti = lax.shift_right_logical(rest, l_shift)
                    li = lax.bitwise_and(rest, L - 1)
                    row = _ds * NT + nti
                    vals = plsc.load_gather(tab_v, [row, li])
                    outds_v[_buf, c, pl.ds(u * _LANES, _LANES)] = vals
                return carry
            lax.fori_loop(0, NT, gather_body, 0)

            @pl.when(has_invalid)
            def _fix_invalid():
                def fix_body(c, carry, _buf=buf):
                    for u in range(L // _LANES):
                        off = c * L + u * _LANES
                        wv = widx_v[pl.ds(off, _LANES)]
                        sl = pl.ds(u * _LANES, _LANES)
                        outds_v[_buf, c, sl] = jnp.where(
                            wv < G * n_dim, outds_v[_buf, c, sl], 0.0)
                    return carry
                lax.fori_loop(0, NT, fix_body, 0)

            oc = pltpu.make_async_copy(outds_v.at[buf],
                                       out5.at[b, dt, :, ds, :], osem)
            oc.start()
            out_copies[buf] = oc
        for oc in out_copies:
            if oc is not None:
                oc.wait()

    @pl.when(jnp.logical_not(single))
    def _slow_path():
        out_copies = [None, None]
        for ds in range(DS):
            stage = [pltpu.make_async_copy(x6.at[b, g, dt, :, ds, :],
                                           tab_v.at[pl.ds(g * NT, NT)], sem)
                     for g in range(G)]
            for g in range(G):
                @pl.when(lax.bitwise_and(
                    lax.shift_right_logical(bits, g), 1) != 0)
                def _start(_c=stage[g]):
                    _c.start()
            for g in range(G):
                @pl.when(lax.bitwise_and(
                    lax.shift_right_logical(bits, g), 1) != 0)
                def _wait(_c=stage[g]):
                    _c.wait()

            buf = ds % 2
            if out_copies[buf] is not None:
                out_copies[buf].wait()

            def gather_body(c, carry, _buf=buf):
                for u in range(L // _LANES):
                    off = c * L + u * _LANES
                    wv = widx_v[pl.ds(off, _LANES)]
                    gi = lax.bitwise_and(
                        lax.shift_right_logical(wv, g_shift), G - 1)
                    rest = lax.bitwise_and(wv, n_dim - 1)
                    nti = lax.shift_right_logical(rest, l_shift)
                    li = lax.bitwise_and(rest, L - 1)
                    vals = plsc.load_gather(tab_v, [gi * NT + nti, li])
                    outds_v[_buf, c, pl.ds(u * _LANES, _LANES)] = vals
                return carry
            lax.fori_loop(0, NT, gather_body, 0)

            @pl.when(has_invalid)
            def _fix_invalid():
                def fix_body(c, carry, _buf=buf):
                    for u in range(L // _LANES):
                        off = c * L + u * _LANES
                        wv = widx_v[pl.ds(off, _LANES)]
                        sl = pl.ds(u * _LANES, _LANES)
                        outds_v[_buf, c, sl] = jnp.where(
                            wv < G * n_dim, outds_v[_buf, c, sl], 0.0)
                    return carry
                lax.fori_loop(0, NT, fix_body, 0)

            oc = pltpu.make_async_copy(outds_v.at[buf],
                                       out5.at[b, dt, :, ds, :], osem)
            oc.start()
            out_copies[buf] = oc
        for oc in out_copies:
            if oc is not None:
                oc.wait()


def kernel(block_features, block_onehot, output_shape):
    B, G, Ng_max, D = block_features.shape
    if block_onehot.ndim == 2:
        block_onehot = block_onehot[None, :, :]
    if block_onehot.shape[0] != B:
        block_onehot = jnp.tile(block_onehot, (B, 1, 1))
    N = block_onehot.shape[1]
    DT, DS, NT, L = D // _SUBL, _SUBL, Ng_max // _TLANE, _TLANE

    oh_t = jnp.transpose(block_onehot, (0, 2, 1))
    oh2 = oh_t.reshape(B * G, N)
    widx, bits = pl.pallas_call(
        functools.partial(_index_kernel, B, G, Ng_max),
        out_shape=[jax.ShapeDtypeStruct((B, 1, N), jnp.int32),
                   jax.ShapeDtypeStruct((B, 1, N), jnp.int32)],
    )(oh2)

    x6 = (block_features.transpose(0, 1, 3, 2)
          .reshape(B, G, DT, DS, NT, L)
          .transpose(0, 1, 2, 4, 3, 5))

    mesh = plsc.VectorSubcoreMesh(core_axis_name="c", subcore_axis_name="s",
                                  num_cores=_NC, num_subcores=_NS)
    out5 = pl.kernel(
        _sc_gather,
        out_type=jax.ShapeDtypeStruct((B, DT, NT, DS, L), jnp.float32),
        mesh=mesh,
        compiler_params=pltpu.CompilerParams(use_tc_tiling_on_sc=False,
                                             needs_layout_passes=False),
        scratch_types=[
            pltpu.VMEM((G * NT, L), jnp.float32),
            pltpu.VMEM((2, NT, L), jnp.float32),
            pltpu.VMEM((N,), jnp.int32),
            pltpu.VMEM((_LANES,), jnp.int32),
            pltpu.SemaphoreType.DMA,
            pltpu.SemaphoreType.DMA,
        ],
    )(x6, widx, bits)
    return out5.transpose(0, 2, 4, 1, 3).reshape(B, N, D)
